# baseline (speedup 1.0000x reference)
import functools

import numpy as np

import jax
import jax.numpy as jnp
from jax import lax
from jax.experimental import pallas as pl
from jax.experimental.pallas import tpu as pltpu
from jax.experimental.pallas import tpu_sc as plsc

N = 10000
NPAD = 10240
F_IN = 256
HID = 512
C = 64
E = 160000
NC = 2
NS = 16
KD = 125
K = 128
NCHUNK = E // KD
_NBUF = 4

_MESH = plsc.VectorSubcoreMesh(
    core_axis_name="c", subcore_axis_name="s", num_cores=NC, num_subcores=NS)
_SC_PARAMS = pltpu.CompilerParams(use_tc_tiling_on_sc=False)

_f32 = jnp.float32


def _fill_zeros(buf, rows, width):
    for r in range(rows):
        for j in range(width // 16):
            buf[r, pl.ds(j * 16, 16)] = jnp.zeros((16,), _f32)


_DEG_W = 16
_DEG_CH = NCHUNK // (NC * NS)


@functools.partial(
    pl.kernel,
    out_type=jax.ShapeDtypeStruct((NPAD, 2 * _DEG_W), _f32),
    mesh=_MESH,
    scratch_types=[
        pltpu.VMEM((_DEG_CH, K), jnp.int32),
        pltpu.VMEM((K, _DEG_W), _f32),
        pltpu.VMEM((128, _DEG_W), _f32),
        pltpu.VMEM_SHARED((NPAD, _DEG_W), _f32),
    ],
    compiler_params=_SC_PARAMS,
)
def _sc_deg(edst_hbm, out_hbm, didx_v, ones_v, zbuf, acc_sh):
    c = lax.axis_index("c")
    s = lax.axis_index("s")
    _fill_zeros(zbuf, 128, _DEG_W)
    for r in range(K):
        ones_v[r, pl.ds(0, 16)] = jnp.ones((16,), _f32)
    for j in range(5):
        pltpu.sync_copy(zbuf, acc_sh.at[pl.ds(s * 640 + j * 128, 128)])
    plsc.subcore_barrier()
    pltpu.sync_copy(edst_hbm.at[pl.ds((c * NS + s) * _DEG_CH, _DEG_CH)],
                    didx_v)

    def body(k, _):
        pltpu.sync_copy(ones_v, acc_sh.at[didx_v.at[k]], add=True)
        return _

    lax.fori_loop(0, _DEG_CH, body, 0)
    plsc.subcore_barrier()
    pltpu.sync_copy(acc_sh.at[pl.ds(s * 640, 640)],
                    out_hbm.at[pl.ds(s * 640, 640),
                               pl.ds(c * _DEG_W, _DEG_W)])


def _make_sc_agg(nch, nph, use_off, name):
    width = 64

    @functools.partial(
        pl.kernel,
        out_type=jax.ShapeDtypeStruct((nph, NPAD, 2 * width), _f32),
        mesh=_MESH,
        scratch_types=(
            [pltpu.VMEM((nch, K), jnp.int32),
             pltpu.VMEM((nch, K), jnp.int32),
             pltpu.VMEM((128, width), _f32)]
            + [pltpu.VMEM((K, width), _f32) for _ in range(_NBUF)]
            + [pltpu.SemaphoreType.DMA for _ in range(_NBUF)]
            + [pltpu.VMEM_SHARED((NPAD, width), _f32)]
        ),
        compiler_params=_SC_PARAMS,
        name=name,
    )
    def agg(table_hbm, esrc_hbm, edst_hbm, out_hbm, gidx_v, didx_v, zbuf,
            *rest):
        bufs = rest[:_NBUF]
        sems = rest[_NBUF:2 * _NBUF]
        acc_sh = rest[2 * _NBUF]
        c = lax.axis_index("c")
        s = lax.axis_index("s")
        _fill_zeros(zbuf, 128, width)

        def zero_own():
            for j in range(5):
                pltpu.sync_copy(zbuf, acc_sh.at[pl.ds(s * 640 + j * 128, 128)])

        zero_own()
        if use_off:
            base = s * nch
        else:
            base = (c * NS + s) * nch
        pltpu.sync_copy(esrc_hbm.at[pl.ds(base, nch)], gidx_v)
        pltpu.sync_copy(edst_hbm.at[pl.ds(base, nch)], didx_v)
        for q in range(nph):
            if use_off:
                off = c * N if q == 0 else jnp.int32(2 * N)
                offv = lax.broadcast(off, (16,))
                for r in range(nch):
                    for j in range(K // 16):
                        sl = pl.ds(j * 16, 16)
                        gidx_v[r, sl] = gidx_v[r, sl] + offv
            plsc.subcore_barrier()
            for b in range(_NBUF - 1):
                pltpu.async_copy(table_hbm.at[gidx_v.at[b]], bufs[b], sems[b])

            def body(j, _):
                for b in range(_NBUF):
                    k = j * _NBUF + b
                    pltpu.make_async_copy(
                        table_hbm.at[gidx_v.at[k]], bufs[b], sems[b]).wait()
                    pltpu.sync_copy(bufs[b], acc_sh.at[didx_v.at[k]],
                                    add=True)
                    nxt = k + _NBUF - 1

                    @pl.when(nxt < nch)
                    def _start():
                        pltpu.async_copy(
                            table_hbm.at[gidx_v.at[nxt]],
                            bufs[(b + _NBUF - 1) % _NBUF],
                            sems[(b + _NBUF - 1) % _NBUF])
                return _

            lax.fori_loop(0, nch // _NBUF, body, 0)
            plsc.subcore_barrier()
            pltpu.sync_copy(acc_sh.at[pl.ds(s * 640, 640)],
                            out_hbm.at[q, pl.ds(s * 640, 640),
                                       pl.ds(c * width, width)])
            if q + 1 < nph:
                zero_own()

    return agg


_sc_agg1 = _make_sc_agg(NCHUNK // NS, 2, True, "sc_agg1")
_sc_agg2 = _make_sc_agg(NCHUNK // (NC * NS), 1, False, "sc_agg2")


def _tc_scale_body(degp_ref, x_ref, dis_ref, x1p_ref):
    q = pl.program_id(0)
    deg = jnp.sum(degp_ref[...], axis=1, keepdims=True) * (1.0 / _DEG_W) + 1.0
    dis = lax.rsqrt(deg)
    dis_ref[...] = dis
    xb = x_ref[...]
    xq = jnp.where((q % 2) == 0, xb[:, :64], xb[:, 64:])
    x1p_ref[...] = xq * dis


def _tc_scale(degp, x):
    blk = 1000
    return pl.pallas_call(
        _tc_scale_body,
        grid=(4, N // blk),
        in_specs=[
            pl.BlockSpec((blk, 2 * _DEG_W), lambda q, i: (i, 0)),
            pl.BlockSpec((blk, 128), lambda q, i: (i, q // 2)),
        ],
        out_specs=[
            pl.BlockSpec((blk, 1), lambda q, i: (i, 0)),
            pl.BlockSpec((blk, 64), lambda q, i: (q * (N // blk) + i, 0)),
        ],
        out_shape=[
            jax.ShapeDtypeStruct((N, 1), _f32),
            jax.ShapeDtypeStruct((4 * N, 64), _f32),
        ],
    )(degp, x)


def _tc_mlp_body(agg_ref, x_ref, dis_ref, W1_ref, b1_ref, W2_ref, p1_ref):
    dis = dis_ref[...]
    agg = jnp.concatenate([agg_ref[0], agg_ref[1]], axis=1)
    ax = dis * agg + (dis * dis) * x_ref[...]
    h = jnp.maximum(
        jnp.dot(ax, W1_ref[...], preferred_element_type=_f32) + b1_ref[...],
        0.0)
    p = jnp.dot(h, W2_ref[...], preferred_element_type=_f32)
    p1_ref[...] = dis * p


def _tc_mlp(agg1, x, dis_col, W1, b1, W2):
    blk = 1000
    return pl.pallas_call(
        _tc_mlp_body,
        grid=(N // blk,),
        in_specs=[
            pl.BlockSpec((2, blk, F_IN // 2), lambda i: (0, i, 0)),
            pl.BlockSpec((blk, F_IN), lambda i: (i, 0)),
            pl.BlockSpec((blk, 1), lambda i: (i, 0)),
            pl.BlockSpec((F_IN, HID), lambda i: (0, 0)),
            pl.BlockSpec((1, HID), lambda i: (0, 0)),
            pl.BlockSpec((HID, C), lambda i: (0, 0)),
        ],
        out_specs=pl.BlockSpec((blk, C), lambda i: (i, 0)),
        out_shape=jax.ShapeDtypeStruct((N, C), _f32),
    )(agg1, x, dis_col, W1, b1, W2)


def _tc_final_body(agg2_ref, p1_ref, dis_ref, b2_ref, logp_ref, z_ref):
    a = agg2_ref[0]
    z = dis_ref[...] * (a[:, :C] + a[:, C:] + p1_ref[...]) + b2_ref[...]
    m = jnp.max(z, axis=1, keepdims=True)
    lse = jnp.log(jnp.sum(jnp.exp(z - m), axis=1, keepdims=True)) + m
    logp_ref[...] = z - lse
    z_ref[...] = z


def _tc_final(agg2, p1, dis_col, b2):
    blk = 1000
    return pl.pallas_call(
        _tc_final_body,
        grid=(N // blk,),
        in_specs=[
            pl.BlockSpec((1, blk, 2 * C), lambda i: (0, i, 0)),
            pl.BlockSpec((blk, C), lambda i: (i, 0)),
            pl.BlockSpec((blk, 1), lambda i: (i, 0)),
            pl.BlockSpec((1, C), lambda i: (0, 0)),
        ],
        out_specs=[
            pl.BlockSpec((blk, C), lambda i: (i, 0)),
            pl.BlockSpec((blk, C), lambda i: (i, 0)),
        ],
        out_shape=[
            jax.ShapeDtypeStruct((N, C), _f32),
            jax.ShapeDtypeStruct((N, C), _f32),
        ],
    )(agg2, p1, dis_col, b2)


_DUMP = (N + (np.arange(NCHUNK * (K - KD), dtype=np.int32)
              % (NPAD - N)).reshape(NCHUNK, K - KD))


def kernel(x, W1, b1, W2, b2, edge_index):
    src = edge_index[0].reshape(NCHUNK, KD)
    dst = edge_index[1].reshape(NCHUNK, KD)
    zpad = jnp.zeros((NCHUNK, K - KD), jnp.int32)
    esrc = jnp.concatenate([src, zpad], axis=1)
    edst = jnp.concatenate([dst, jnp.asarray(_DUMP)], axis=1)

    degp = _sc_deg(edst)
    dis_col, x1p = _tc_scale(degp, x)
    agg1 = _sc_agg1(x1p, esrc, edst)
    p1 = _tc_mlp(agg1, x, dis_col, W1, b1.reshape(1, HID), W2)
    agg2 = _sc_agg2(p1, esrc, edst)
    logp, z = _tc_final(agg2, p1, dis_col, b2.reshape(1, C))
    return (logp, z)


# speedup vs baseline: 1.0088x; 1.0088x over previous
import functools

import numpy as np

import jax
import jax.numpy as jnp
from jax import lax
from jax.experimental import pallas as pl
from jax.experimental.pallas import tpu as pltpu
from jax.experimental.pallas import tpu_sc as plsc

N = 10000
NPAD = 10240
F_IN = 256
HID = 512
C = 64
E = 160000
NC = 2
NS = 16
KD = 125
K = 128
NCHUNK = E // KD
_NBUF = 4

_MESH = plsc.VectorSubcoreMesh(
    core_axis_name="c", subcore_axis_name="s", num_cores=NC, num_subcores=NS)
_SC_PARAMS = pltpu.CompilerParams(use_tc_tiling_on_sc=False)

_f32 = jnp.float32


def _fill_zeros(buf, rows, width):
    for r in range(rows):
        for j in range(width // 16):
            buf[r, pl.ds(j * 16, 16)] = jnp.zeros((16,), _f32)


_DEG_W = 16
_DEG_CH = NCHUNK // (NC * NS)


@functools.partial(
    pl.kernel,
    out_type=jax.ShapeDtypeStruct((NC, NPAD, _DEG_W), _f32),
    mesh=_MESH,
    scratch_types=[
        pltpu.VMEM((_DEG_CH, K), jnp.int32),
        pltpu.VMEM((K, _DEG_W), _f32),
        pltpu.VMEM((128, _DEG_W), _f32),
        pltpu.VMEM_SHARED((NPAD, _DEG_W), _f32),
    ],
    compiler_params=_SC_PARAMS,
)
def _sc_deg(edst_hbm, out_hbm, didx_v, ones_v, zbuf, acc_sh):
    c = lax.axis_index("c")
    s = lax.axis_index("s")
    _fill_zeros(zbuf, 128, _DEG_W)
    for r in range(K):
        ones_v[r, pl.ds(0, 16)] = jnp.ones((16,), _f32)
    for j in range(5):
        pltpu.sync_copy(zbuf, acc_sh.at[pl.ds(s * 640 + j * 128, 128)])
    plsc.subcore_barrier()
    pltpu.sync_copy(edst_hbm.at[c * NS + s], didx_v)

    def body(k, _):
        pltpu.sync_copy(ones_v, acc_sh.at[didx_v.at[k]], add=True)
        return _

    lax.fori_loop(0, _DEG_CH, body, 0)
    plsc.subcore_barrier()
    pltpu.sync_copy(acc_sh.at[pl.ds(s * 640, 640)],
                    out_hbm.at[c, pl.ds(s * 640, 640)])


def _make_sc_agg(nch, nph, use_off, name):
    width = 64

    @functools.partial(
        pl.kernel,
        out_type=jax.ShapeDtypeStruct((nph, NPAD, 2 * width), _f32),
        mesh=_MESH,
        scratch_types=(
            [pltpu.VMEM((nch, K), jnp.int32),
             pltpu.VMEM((nch, K), jnp.int32),
             pltpu.VMEM((128, width), _f32)]
            + [pltpu.VMEM((K, width), _f32) for _ in range(_NBUF)]
            + [pltpu.SemaphoreType.DMA for _ in range(_NBUF)]
            + [pltpu.VMEM_SHARED((NPAD, width), _f32)]
        ),
        compiler_params=_SC_PARAMS,
        name=name,
    )
    def agg(table_hbm, esrc_hbm, edst_hbm, out_hbm, gidx_v, didx_v, zbuf,
            *rest):
        bufs = rest[:_NBUF]
        sems = rest[_NBUF:2 * _NBUF]
        acc_sh = rest[2 * _NBUF]
        c = lax.axis_index("c")
        s = lax.axis_index("s")
        _fill_zeros(zbuf, 128, width)

        def zero_own():
            for j in range(5):
                pltpu.sync_copy(zbuf, acc_sh.at[pl.ds(s * 640 + j * 128, 128)])

        zero_own()
        w = s if use_off else c * NS + s
        pltpu.sync_copy(esrc_hbm.at[w], gidx_v)
        pltpu.sync_copy(edst_hbm.at[w], didx_v)
        for q in range(nph):
            if use_off:
                off = c * N if q == 0 else jnp.int32(2 * N)
                offv = lax.broadcast(off, (16,))
                for r in range(nch):
                    for j in range(K // 16):
                        sl = pl.ds(j * 16, 16)
                        gidx_v[r, sl] = gidx_v[r, sl] + offv
            plsc.subcore_barrier()
            for b in range(_NBUF - 1):
                pltpu.async_copy(table_hbm.at[gidx_v.at[b]], bufs[b], sems[b])

            def body(j, _):
                for b in range(_NBUF):
                    k = j * _NBUF + b
                    pltpu.make_async_copy(
                        table_hbm.at[gidx_v.at[k]], bufs[b], sems[b]).wait()
                    pltpu.sync_copy(bufs[b], acc_sh.at[didx_v.at[k]],
                                    add=True)
                    nxt = k + _NBUF - 1

                    @pl.when(nxt < nch)
                    def _start():
                        pltpu.async_copy(
                            table_hbm.at[gidx_v.at[nxt]],
                            bufs[(b + _NBUF - 1) % _NBUF],
                            sems[(b + _NBUF - 1) % _NBUF])
                return _

            lax.fori_loop(0, nch // _NBUF, body, 0)
            plsc.subcore_barrier()
            pltpu.sync_copy(acc_sh.at[pl.ds(s * 640, 640)],
                            out_hbm.at[q, pl.ds(s * 640, 640),
                                       pl.ds(c * width, width)])
            if q + 1 < nph:
                zero_own()

    return agg


_sc_agg1 = _make_sc_agg(NCHUNK // NS, 2, True, "sc_agg1")
_sc_agg2 = _make_sc_agg(NCHUNK // (NC * NS), 1, False, "sc_agg2")


def _tc_scale_body(degp_ref, x_ref, dis_ref, *out_refs):
    degs = jnp.sum(degp_ref[...], axis=0)
    deg = jnp.sum(degs, axis=1, keepdims=True) * (1.0 / _DEG_W) + 1.0
    dis = lax.rsqrt(deg)
    dis_ref[...] = dis
    x1 = x_ref[...] * dis
    for q, o_ref in enumerate(out_refs):
        o_ref[...] = x1[:, q * 64:(q + 1) * 64]


def _tc_scale(degp, x):
    blk = 1000
    return pl.pallas_call(
        _tc_scale_body,
        grid=(N // blk,),
        in_specs=[
            pl.BlockSpec((NC, blk, _DEG_W), lambda i: (0, i, 0)),
            pl.BlockSpec((blk, F_IN), lambda i: (i, 0)),
        ],
        out_specs=[pl.BlockSpec((blk, 1), lambda i: (i, 0))]
        + [pl.BlockSpec((blk, 64), lambda i: (i, 0))] * 4,
        out_shape=[jax.ShapeDtypeStruct((N, 1), _f32)]
        + [jax.ShapeDtypeStruct((N, 64), _f32)] * 4,
    )(degp, x)


def _tc_mlp_body(agg_ref, x_ref, dis_ref, W1_ref, b1_ref, W2_ref, p1_ref):
    dis = dis_ref[...]
    agg = jnp.concatenate([agg_ref[0], agg_ref[1]], axis=1)
    ax = dis * agg + (dis * dis) * x_ref[...]
    h = jnp.maximum(
        jnp.dot(ax, W1_ref[...], preferred_element_type=_f32) + b1_ref[...],
        0.0)
    p = jnp.dot(h, W2_ref[...], preferred_element_type=_f32)
    p1_ref[...] = dis * p


def _tc_mlp(agg1, x, dis_col, W1, b1, W2):
    blk = 1000
    return pl.pallas_call(
        _tc_mlp_body,
        grid=(N // blk,),
        in_specs=[
            pl.BlockSpec((2, blk, F_IN // 2), lambda i: (0, i, 0)),
            pl.BlockSpec((blk, F_IN), lambda i: (i, 0)),
            pl.BlockSpec((blk, 1), lambda i: (i, 0)),
            pl.BlockSpec((F_IN, HID), lambda i: (0, 0)),
            pl.BlockSpec((1, HID), lambda i: (0, 0)),
            pl.BlockSpec((HID, C), lambda i: (0, 0)),
        ],
        out_specs=pl.BlockSpec((blk, C), lambda i: (i, 0)),
        out_shape=jax.ShapeDtypeStruct((N, C), _f32),
    )(agg1, x, dis_col, W1, b1, W2)


def _tc_final_body(agg2_ref, p1_ref, dis_ref, b2_ref, logp_ref, z_ref):
    a = agg2_ref[0]
    z = dis_ref[...] * (a[:, :C] + a[:, C:] + p1_ref[...]) + b2_ref[...]
    m = jnp.max(z, axis=1, keepdims=True)
    lse = jnp.log(jnp.sum(jnp.exp(z - m), axis=1, keepdims=True)) + m
    logp_ref[...] = z - lse
    z_ref[...] = z


def _tc_final(agg2, p1, dis_col, b2):
    blk = 1000
    return pl.pallas_call(
        _tc_final_body,
        grid=(N // blk,),
        in_specs=[
            pl.BlockSpec((1, blk, 2 * C), lambda i: (0, i, 0)),
            pl.BlockSpec((blk, C), lambda i: (i, 0)),
            pl.BlockSpec((blk, 1), lambda i: (i, 0)),
            pl.BlockSpec((1, C), lambda i: (0, 0)),
        ],
        out_specs=[
            pl.BlockSpec((blk, C), lambda i: (i, 0)),
            pl.BlockSpec((blk, C), lambda i: (i, 0)),
        ],
        out_shape=[
            jax.ShapeDtypeStruct((N, C), _f32),
            jax.ShapeDtypeStruct((N, C), _f32),
        ],
    )(agg2, p1, dis_col, b2)


_DUMP = (N + (np.arange(NCHUNK * (K - KD), dtype=np.int32)
              % (NPAD - N)).reshape(NCHUNK, K - KD))


def kernel(x, W1, b1, W2, b2, edge_index):
    src = edge_index[0].reshape(NCHUNK, KD)
    dst = edge_index[1].reshape(NCHUNK, KD)
    zpad = jnp.zeros((NCHUNK, K - KD), jnp.int32)
    esrc = jnp.concatenate([src, zpad], axis=1)
    edst = jnp.concatenate([dst, jnp.asarray(_DUMP)], axis=1)
    esrc1 = esrc.reshape(NS, NCHUNK // NS, K)
    edst1 = edst.reshape(NS, NCHUNK // NS, K)
    nch2 = NCHUNK // (NC * NS)
    esrc2 = esrc.reshape(NC * NS, nch2, K)
    edst2 = edst.reshape(NC * NS, nch2, K)

    degp = _sc_deg(edst2)
    dis_col, *x1q = _tc_scale(degp, x)
    x1p = jnp.concatenate(x1q, axis=0)
    agg1 = _sc_agg1(x1p, esrc1, edst1)
    p1 = _tc_mlp(agg1, x, dis_col, W1, b1.reshape(1, HID), W2)
    agg2 = _sc_agg2(p1, esrc2, edst2)
    logp, z = _tc_final(agg2, p1, dis_col, b2.reshape(1, C))
    return (logp, z)


# speedup vs baseline: 1.7560x; 1.7407x over previous
import functools

import numpy as np

import jax
import jax.numpy as jnp
from jax import lax
from jax.experimental import pallas as pl
from jax.experimental.pallas import tpu as pltpu
from jax.experimental.pallas import tpu_sc as plsc

N = 10000
NPAD = 10240
F_IN = 256
HID = 512
C = 64
E = 160000
NC = 2
NS = 16
KD = 125
K = 125
NCHUNK = E // KD
_NBUF = 4

_MESH = plsc.VectorSubcoreMesh(
    core_axis_name="c", subcore_axis_name="s", num_cores=NC, num_subcores=NS)
_SC_PARAMS = pltpu.CompilerParams(use_tc_tiling_on_sc=False)

_f32 = jnp.float32


def _fill_zeros(buf, rows, width):
    for r in range(rows):
        for j in range(width // 16):
            buf[r, pl.ds(j * 16, 16)] = jnp.zeros((16,), _f32)


_DEG_W = 16
_DEG_CH = NCHUNK // (NC * NS)


@functools.partial(
    pl.kernel,
    out_type=jax.ShapeDtypeStruct((NC, NPAD, _DEG_W), _f32),
    mesh=_MESH,
    scratch_types=[
        pltpu.VMEM((_DEG_CH, K), jnp.int32),
        pltpu.VMEM((K, _DEG_W), _f32),
        pltpu.VMEM((128, _DEG_W), _f32),
        pltpu.VMEM_SHARED((NPAD, _DEG_W), _f32),
    ],
    compiler_params=_SC_PARAMS,
)
def _sc_deg(edst_hbm, out_hbm, didx_v, ones_v, zbuf, acc_sh):
    c = lax.axis_index("c")
    s = lax.axis_index("s")
    _fill_zeros(zbuf, 128, _DEG_W)
    for r in range(K):
        ones_v[r, pl.ds(0, 16)] = jnp.ones((16,), _f32)
    for j in range(5):
        pltpu.sync_copy(zbuf, acc_sh.at[pl.ds(s * 640 + j * 128, 128)])
    plsc.subcore_barrier()
    pltpu.sync_copy(edst_hbm.at[c * NS + s], didx_v)

    def body(k, _):
        pltpu.sync_copy(ones_v, acc_sh.at[didx_v.at[k]], add=True)
        return _

    lax.fori_loop(0, _DEG_CH, body, 0)
    plsc.subcore_barrier()
    pltpu.sync_copy(acc_sh.at[pl.ds(s * 640, 640)],
                    out_hbm.at[c, pl.ds(s * 640, 640)])


def _make_sc_agg(nch, nph, use_off, name):
    width = 64

    @functools.partial(
        pl.kernel,
        out_type=jax.ShapeDtypeStruct((nph, NPAD, 2 * width), _f32),
        mesh=_MESH,
        scratch_types=(
            [pltpu.VMEM((nch, K), jnp.int32),
             pltpu.VMEM((nch, K), jnp.int32),
             pltpu.VMEM((128, width), _f32)]
            + [pltpu.VMEM((K, width), _f32) for _ in range(_NBUF)]
            + [pltpu.SemaphoreType.DMA for _ in range(_NBUF)]
            + [pltpu.VMEM_SHARED((NPAD, width), _f32)]
        ),
        compiler_params=_SC_PARAMS,
        name=name,
    )
    def agg(table_hbm, esrc_hbm, edst_hbm, out_hbm, gidx_v, didx_v, zbuf,
            *rest):
        bufs = rest[:_NBUF]
        sems = rest[_NBUF:2 * _NBUF]
        acc_sh = rest[2 * _NBUF]
        c = lax.axis_index("c")
        s = lax.axis_index("s")
        _fill_zeros(zbuf, 128, width)

        def zero_own():
            for j in range(5):
                pltpu.sync_copy(zbuf, acc_sh.at[pl.ds(s * 640 + j * 128, 128)])

        zero_own()
        w = s if use_off else c * NS + s
        pltpu.sync_copy(esrc_hbm.at[w], gidx_v)
        pltpu.sync_copy(edst_hbm.at[w], didx_v)
        for q in range(nph):
            if use_off:
                off = c * N if q == 0 else jnp.int32(2 * N)
                offv = lax.broadcast(off, (16,))
                for r in range(nch):
                    for j in range(K // 16):
                        sl = pl.ds(j * 16, 16)
                        gidx_v[r, sl] = gidx_v[r, sl] + offv
            plsc.subcore_barrier()
            for b in range(_NBUF - 1):
                pltpu.async_copy(table_hbm.at[gidx_v.at[b]], bufs[b], sems[b])

            def body(j, _):
                for b in range(_NBUF):
                    k = j * _NBUF + b
                    pltpu.make_async_copy(
                        table_hbm.at[gidx_v.at[k]], bufs[b], sems[b]).wait()
                    pltpu.sync_copy(bufs[b], acc_sh.at[didx_v.at[k]],
                                    add=True)
                    nxt = k + _NBUF - 1

                    @pl.when(nxt < nch)
                    def _start():
                        pltpu.async_copy(
                            table_hbm.at[gidx_v.at[nxt]],
                            bufs[(b + _NBUF - 1) % _NBUF],
                            sems[(b + _NBUF - 1) % _NBUF])
                return _

            lax.fori_loop(0, nch // _NBUF, body, 0)
            plsc.subcore_barrier()
            pltpu.sync_copy(acc_sh.at[pl.ds(s * 640, 640)],
                            out_hbm.at[q, pl.ds(s * 640, 640),
                                       pl.ds(c * width, width)])
            if q + 1 < nph:
                zero_own()

    return agg


_sc_agg1 = _make_sc_agg(NCHUNK // NS, 2, True, "sc_agg1")
_sc_agg2 = _make_sc_agg(NCHUNK // (NC * NS), 1, False, "sc_agg2")


def _tc_scale_body(degp_ref, x_ref, dis_ref, *out_refs):
    degs = jnp.sum(degp_ref[...], axis=0)
    deg = jnp.sum(degs, axis=1, keepdims=True) * (1.0 / _DEG_W) + 1.0
    dis = lax.rsqrt(deg)
    dis_ref[...] = dis
    x1 = x_ref[...] * dis
    for q, o_ref in enumerate(out_refs):
        o_ref[...] = x1[:, q * 64:(q + 1) * 64]


def _tc_scale(degp, x):
    blk = 1000
    return pl.pallas_call(
        _tc_scale_body,
        grid=(N // blk,),
        in_specs=[
            pl.BlockSpec((NC, blk, _DEG_W), lambda i: (0, i, 0)),
            pl.BlockSpec((blk, F_IN), lambda i: (i, 0)),
        ],
        out_specs=[pl.BlockSpec((blk, 1), lambda i: (i, 0))]
        + [pl.BlockSpec((blk, 64), lambda i: (i, 0))] * 4,
        out_shape=[jax.ShapeDtypeStruct((N, 1), _f32)]
        + [jax.ShapeDtypeStruct((N, 64), _f32)] * 4,
    )(degp, x)


def _tc_mlp_body(agg_ref, x_ref, dis_ref, W1_ref, b1_ref, W2_ref, p1_ref):
    dis = dis_ref[...]
    agg = jnp.concatenate([agg_ref[0], agg_ref[1]], axis=1)
    ax = dis * agg + (dis * dis) * x_ref[...]
    h = jnp.maximum(
        jnp.dot(ax, W1_ref[...], preferred_element_type=_f32) + b1_ref[...],
        0.0)
    p = jnp.dot(h, W2_ref[...], preferred_element_type=_f32)
    p1_ref[...] = dis * p


def _tc_mlp(agg1, x, dis_col, W1, b1, W2):
    blk = 1000
    return pl.pallas_call(
        _tc_mlp_body,
        grid=(N // blk,),
        in_specs=[
            pl.BlockSpec((2, blk, F_IN // 2), lambda i: (0, i, 0)),
            pl.BlockSpec((blk, F_IN), lambda i: (i, 0)),
            pl.BlockSpec((blk, 1), lambda i: (i, 0)),
            pl.BlockSpec((F_IN, HID), lambda i: (0, 0)),
            pl.BlockSpec((1, HID), lambda i: (0, 0)),
            pl.BlockSpec((HID, C), lambda i: (0, 0)),
        ],
        out_specs=pl.BlockSpec((blk, C), lambda i: (i, 0)),
        out_shape=jax.ShapeDtypeStruct((N, C), _f32),
    )(agg1, x, dis_col, W1, b1, W2)


def _tc_final_body(agg2_ref, p1_ref, dis_ref, b2_ref, logp_ref, z_ref):
    a = agg2_ref[0]
    z = dis_ref[...] * (a[:, :C] + a[:, C:] + p1_ref[...]) + b2_ref[...]
    m = jnp.max(z, axis=1, keepdims=True)
    lse = jnp.log(jnp.sum(jnp.exp(z - m), axis=1, keepdims=True)) + m
    logp_ref[...] = z - lse
    z_ref[...] = z


def _tc_final(agg2, p1, dis_col, b2):
    blk = 1000
    return pl.pallas_call(
        _tc_final_body,
        grid=(N // blk,),
        in_specs=[
            pl.BlockSpec((1, blk, 2 * C), lambda i: (0, i, 0)),
            pl.BlockSpec((blk, C), lambda i: (i, 0)),
            pl.BlockSpec((blk, 1), lambda i: (i, 0)),
            pl.BlockSpec((1, C), lambda i: (0, 0)),
        ],
        out_specs=[
            pl.BlockSpec((blk, C), lambda i: (i, 0)),
            pl.BlockSpec((blk, C), lambda i: (i, 0)),
        ],
        out_shape=[
            jax.ShapeDtypeStruct((N, C), _f32),
            jax.ShapeDtypeStruct((N, C), _f32),
        ],
    )(agg2, p1, dis_col, b2)


_DUMP = (N + (np.arange(NCHUNK * max(K - KD, 1), dtype=np.int32)
              % (NPAD - N)).reshape(NCHUNK, max(K - KD, 1)))


def kernel(x, W1, b1, W2, b2, edge_index):
    src = edge_index[0].reshape(NCHUNK, KD)
    dst = edge_index[1].reshape(NCHUNK, KD)
    if K > KD:
        zpad = jnp.zeros((NCHUNK, K - KD), jnp.int32)
        esrc = jnp.concatenate([src, zpad], axis=1)
        edst = jnp.concatenate([dst, jnp.asarray(_DUMP)], axis=1)
    else:
        esrc, edst = src, dst
    esrc1 = esrc.reshape(NS, NCHUNK // NS, K)
    edst1 = edst.reshape(NS, NCHUNK // NS, K)
    nch2 = NCHUNK // (NC * NS)
    esrc2 = esrc.reshape(NC * NS, nch2, K)
    edst2 = edst.reshape(NC * NS, nch2, K)

    degp = _sc_deg(edst2)
    dis_col, *x1q = _tc_scale(degp, x)
    x1p = jnp.concatenate(x1q, axis=0)
    agg1 = _sc_agg1(x1p, esrc1, edst1)
    p1 = _tc_mlp(agg1, x, dis_col, W1, b1.reshape(1, HID), W2)
    agg2 = _sc_agg2(p1, esrc2, edst2)
    logp, z = _tc_final(agg2, p1, dis_col, b2.reshape(1, C))
    return (logp, z)
